# baseline (device time: 45728 ns/iter reference)
import pathlib

import jax
import jax.numpy as jnp
from jax import lax
from jax.experimental import pallas as pl
from jax.experimental.pallas import tpu as pltpu

N_DEV = 16

try:
    ABL = set(pathlib.Path(__file__).with_name("abl.txt").read_text().split())
except OSError:
    ABL = set()
NOCOMM = "nocomm" in ABL
NOGEMM = "nogemm" in ABL
NOAMAX = "noamax" in ABL or NOCOMM
NOQUANT = "noquant" in ABL
NOXFER = "noxfer" in ABL or NOCOMM


def kernel(x, w_mat):
    m_glob, k_blk = x.shape
    k_glob, n = w_mat.shape
    blk = m_glob // N_DEV

    def body(x_hbm_ref, w_hbm_ref, out_ref, xbf_ref, w_ref, xrows_ref,
             xstage_ref, amax_ref, send_sems, recv_sems, amax_send_sems,
             amax_recv_sems, credit_sems, in_copy_sems):
        my = lax.axis_index("i")

        x_copy = pltpu.make_async_copy(x_hbm_ref, xbf_ref, in_copy_sems.at[0])
        w_copy = pltpu.make_async_copy(w_hbm_ref, w_ref, in_copy_sems.at[1])
        x_copy.start()
        w_copy.start()

        x_rdmas = []
        if not NOCOMM:
            bar = pltpu.get_barrier_semaphore()
            pl.semaphore_signal(bar, inc=1, device_id=(my,),
                                device_id_type=pl.DeviceIdType.MESH)
            pl.semaphore_wait(bar, 1)

            for k in range(1, N_DEV):
                peer = lax.rem(my + k, N_DEV)
                pl.semaphore_signal(credit_sems.at[my], inc=1,
                                    device_id=(peer,),
                                    device_id_type=pl.DeviceIdType.MESH)

        x_copy.wait()

        n_grp = 4
        grp = N_DEV // n_grp
        my_grp = lax.div(my, grp)

        if not NOCOMM:
            for idx in (range(N_DEV) if not NOXFER else []):
                jj, r = divmod(idx, grp)
                dst = lax.rem(my_grp - jj + n_grp, n_grp) * grp + r
                rdma = pltpu.make_async_remote_copy(
                    src_ref=xbf_ref.at[pl.ds(dst * blk, blk), :],
                    dst_ref=xrows_ref.at[my],
                    send_sem=send_sems.at[idx],
                    recv_sem=recv_sems.at[my],
                    device_id=(dst,),
                    device_id_type=pl.DeviceIdType.MESH,
                )

                @pl.when(dst != my)
                def _():
                    pl.semaphore_wait(credit_sems.at[dst], 1)
                    rdma.start()

                x_rdmas.append((dst, rdma))

        dot = lambda a, b: jax.lax.dot_general(
            a, b, (((1,), (0,)), ((), ())),
            precision=lax.Precision.DEFAULT,
            preferred_element_type=jnp.float32,
        )
        w_copy.wait()
        if not NOGEMM:
            out_ref[...] = dot(xbf_ref[pl.ds(my * blk, blk), :].astype(
                jnp.float32), w_ref[pl.ds(my * blk, blk), :])
        else:
            out_ref[...] = jnp.zeros((blk, n), jnp.float32)
        for j in range(n_grp):
            g = lax.rem(my_grp + j, n_grp)
            for r in range(grp):
                s = g * grp + r

                if not (NOCOMM or NOXFER):
                    @pl.when(s != my)
                    def _():
                        recv = pltpu.make_async_remote_copy(
                            src_ref=xrows_ref.at[s],
                            dst_ref=xrows_ref.at[s],
                            send_sem=send_sems.at[0],
                            recv_sem=recv_sems.at[s],
                            device_id=(my,),
                            device_id_type=pl.DeviceIdType.MESH,
                        )
                        recv.wait_recv()

                if not NOGEMM:
                    @pl.when(s != my)
                    def _():
                        xstage_ref[:, r * k_blk:(r + 1) * k_blk] = (
                            xrows_ref[s].astype(jnp.float32))

                    @pl.when(s == my)
                    def _():
                        xstage_ref[:, r * k_blk:(r + 1) * k_blk] = (
                            jnp.zeros((blk, k_blk), jnp.float32))
            if NOGEMM:
                continue
            out_ref[...] += dot(xstage_ref[...],
                                w_ref[pl.ds(g * grp * blk, grp * blk), :])

        local_amax = jnp.max(jnp.abs(out_ref[...]))
        amax_rdmas = []
        if not NOAMAX:
            amax_ref[pl.ds(my, 1), :, :] = jnp.full((1, 8, 128), local_amax,
                                                    jnp.float32)
            for k in range(1, N_DEV):
                dst = lax.rem(my + k, N_DEV)
                rdma = pltpu.make_async_remote_copy(
                    src_ref=amax_ref.at[my],
                    dst_ref=amax_ref.at[my],
                    send_sem=amax_send_sems.at[k],
                    recv_sem=amax_recv_sems.at[my],
                    device_id=(dst,),
                    device_id_type=pl.DeviceIdType.MESH,
                )
                rdma.start()
                amax_rdmas.append(rdma)
            for k in range(1, N_DEV):
                s = lax.rem(my + k, N_DEV)
                recv = pltpu.make_async_remote_copy(
                    src_ref=amax_ref.at[s],
                    dst_ref=amax_ref.at[s],
                    send_sem=amax_send_sems.at[0],
                    recv_sem=amax_recv_sems.at[s],
                    device_id=(my,),
                    device_id_type=pl.DeviceIdType.MESH,
                )
                recv.wait_recv()
            gmax = jnp.max(amax_ref[...])
        else:
            gmax = local_amax

        if not NOQUANT:
            scale = jnp.maximum(gmax, 1e-20) / 127.0
            q = jnp.clip(jnp.round(out_ref[...] / scale), -127.0, 127.0)
            out_ref[...] = q * scale

        for dst, r in x_rdmas:
            @pl.when(dst != my)
            def _():
                r.wait_send()
        for r in amax_rdmas:
            r.wait_send()

    return pl.pallas_call(
        body,
        out_shape=jax.ShapeDtypeStruct((blk, n), jnp.float32),
        in_specs=[
            pl.BlockSpec(memory_space=pl.ANY),
            pl.BlockSpec(memory_space=pl.ANY),
        ],
        out_specs=pl.BlockSpec(memory_space=pltpu.VMEM),
        scratch_shapes=[
            pltpu.VMEM((m_glob, k_blk), jnp.bfloat16),
            pltpu.VMEM((k_glob, n), jnp.float32),
            pltpu.VMEM((N_DEV, blk, k_blk), jnp.bfloat16),
            pltpu.VMEM((blk, 4 * k_blk), jnp.float32),
            pltpu.VMEM((N_DEV, 8, 128), jnp.float32),
            pltpu.SemaphoreType.DMA((N_DEV,)),
            pltpu.SemaphoreType.DMA((N_DEV,)),
            pltpu.SemaphoreType.DMA((N_DEV,)),
            pltpu.SemaphoreType.DMA((N_DEV,)),
            pltpu.SemaphoreType.REGULAR((N_DEV,)),
            pltpu.SemaphoreType.DMA((2,)),
        ],
        compiler_params=pltpu.CompilerParams(
            collective_id=None if NOCOMM else 0,
            vmem_limit_bytes=100 * 1024 * 1024,
        ),
    )(x.astype(jnp.bfloat16), w_mat)


# device time: 42477 ns/iter; 1.0765x vs baseline; 1.0765x over previous
import pathlib

import jax
import jax.numpy as jnp
from jax import lax
from jax.experimental import pallas as pl
from jax.experimental.pallas import tpu as pltpu

N_DEV = 16

try:
    ABL = set(pathlib.Path(__file__).with_name("abl.txt").read_text().split())
except OSError:
    ABL = set()
NOCOMM = "nocomm" in ABL
NOGEMM = "nogemm" in ABL
NOAMAX = "noamax" in ABL or NOCOMM
NOQUANT = "noquant" in ABL
NOXFER = "noxfer" in ABL or NOCOMM


def kernel(x, w_mat):
    m_glob, k_blk = x.shape
    k_glob, n = w_mat.shape
    blk = m_glob // N_DEV

    def body(x_hbm_ref, w_hbm_ref, out_ref, x_ref, w_ref, xbf_ref, xrows_ref,
             xstage_ref, amax_ref, send_sems, recv_sems, amax_send_sems,
             amax_recv_sems, credit_sems, in_copy_sems):
        my = lax.axis_index("i")

        x_copy = pltpu.make_async_copy(x_hbm_ref, x_ref, in_copy_sems.at[0])
        w_copy = pltpu.make_async_copy(w_hbm_ref, w_ref, in_copy_sems.at[1])
        x_copy.start()
        w_copy.start()

        x_rdmas = []
        if not NOCOMM:
            bar = pltpu.get_barrier_semaphore()
            pl.semaphore_signal(bar, inc=1, device_id=(my,),
                                device_id_type=pl.DeviceIdType.MESH)
            pl.semaphore_wait(bar, 1)

            for k in range(1, N_DEV):
                peer = lax.rem(my + k, N_DEV)
                pl.semaphore_signal(credit_sems.at[my], inc=1,
                                    device_id=(peer,),
                                    device_id_type=pl.DeviceIdType.MESH)

        x_copy.wait()
        xbf_ref[...] = x_ref[...].astype(jnp.bfloat16)

        n_grp = 4
        grp = N_DEV // n_grp
        my_grp = lax.div(my, grp)

        if not NOCOMM:
            for idx in (range(N_DEV) if not NOXFER else []):
                jj, r = divmod(idx, grp)
                dst = lax.rem(my_grp - jj + n_grp, n_grp) * grp + r
                rdma = pltpu.make_async_remote_copy(
                    src_ref=xbf_ref.at[pl.ds(dst * blk, blk), :],
                    dst_ref=xrows_ref.at[my],
                    send_sem=send_sems.at[idx],
                    recv_sem=recv_sems.at[my],
                    device_id=(dst,),
                    device_id_type=pl.DeviceIdType.MESH,
                )

                @pl.when(dst != my)
                def _():
                    pl.semaphore_wait(credit_sems.at[dst], 1)
                    rdma.start()

                x_rdmas.append((dst, rdma))

        dot = lambda a, b: jax.lax.dot_general(
            a, b, (((1,), (0,)), ((), ())),
            precision=lax.Precision.DEFAULT,
            preferred_element_type=jnp.float32,
        )
        w_copy.wait()
        if not NOGEMM:
            out_ref[...] = dot(x_ref[pl.ds(my * blk, blk), :],
                               w_ref[pl.ds(my * blk, blk), :])
        else:
            out_ref[...] = jnp.zeros((blk, n), jnp.float32)
        for j in range(n_grp):
            g = lax.rem(my_grp + j, n_grp)
            for r in range(grp):
                s = g * grp + r

                if not (NOCOMM or NOXFER):
                    @pl.when(s != my)
                    def _():
                        recv = pltpu.make_async_remote_copy(
                            src_ref=xrows_ref.at[s],
                            dst_ref=xrows_ref.at[s],
                            send_sem=send_sems.at[0],
                            recv_sem=recv_sems.at[s],
                            device_id=(my,),
                            device_id_type=pl.DeviceIdType.MESH,
                        )
                        recv.wait_recv()

                if not NOGEMM:
                    @pl.when(s != my)
                    def _():
                        xstage_ref[:, r * k_blk:(r + 1) * k_blk] = (
                            xrows_ref[s].astype(jnp.float32))

                    @pl.when(s == my)
                    def _():
                        xstage_ref[:, r * k_blk:(r + 1) * k_blk] = (
                            jnp.zeros((blk, k_blk), jnp.float32))
            if NOGEMM:
                continue
            out_ref[...] += dot(xstage_ref[...],
                                w_ref[pl.ds(g * grp * blk, grp * blk), :])

        local_amax = jnp.max(jnp.abs(out_ref[...]))
        amax_rdmas = []
        if not NOAMAX:
            amax_ref[pl.ds(my, 1), :, :] = jnp.full((1, 8, 128), local_amax,
                                                    jnp.float32)
            for k in range(1, N_DEV):
                dst = lax.rem(my + k, N_DEV)
                rdma = pltpu.make_async_remote_copy(
                    src_ref=amax_ref.at[my],
                    dst_ref=amax_ref.at[my],
                    send_sem=amax_send_sems.at[k],
                    recv_sem=amax_recv_sems.at[my],
                    device_id=(dst,),
                    device_id_type=pl.DeviceIdType.MESH,
                )
                rdma.start()
                amax_rdmas.append(rdma)
            for k in range(1, N_DEV):
                s = lax.rem(my + k, N_DEV)
                recv = pltpu.make_async_remote_copy(
                    src_ref=amax_ref.at[s],
                    dst_ref=amax_ref.at[s],
                    send_sem=amax_send_sems.at[0],
                    recv_sem=amax_recv_sems.at[s],
                    device_id=(my,),
                    device_id_type=pl.DeviceIdType.MESH,
                )
                recv.wait_recv()
            gmax = jnp.max(amax_ref[...])
        else:
            gmax = local_amax

        if not NOQUANT:
            scale = jnp.maximum(gmax, 1e-20) / 127.0
            q = jnp.clip(jnp.round(out_ref[...] / scale), -127.0, 127.0)
            out_ref[...] = q * scale

        for dst, r in x_rdmas:
            @pl.when(dst != my)
            def _():
                r.wait_send()
        for r in amax_rdmas:
            r.wait_send()

    return pl.pallas_call(
        body,
        out_shape=jax.ShapeDtypeStruct((blk, n), jnp.float32),
        in_specs=[
            pl.BlockSpec(memory_space=pl.ANY),
            pl.BlockSpec(memory_space=pl.ANY),
        ],
        out_specs=pl.BlockSpec(memory_space=pltpu.VMEM),
        scratch_shapes=[
            pltpu.VMEM((m_glob, k_blk), jnp.float32),
            pltpu.VMEM((k_glob, n), jnp.float32),
            pltpu.VMEM((m_glob, k_blk), jnp.bfloat16),
            pltpu.VMEM((N_DEV, blk, k_blk), jnp.bfloat16),
            pltpu.VMEM((blk, 4 * k_blk), jnp.float32),
            pltpu.VMEM((N_DEV, 8, 128), jnp.float32),
            pltpu.SemaphoreType.DMA((N_DEV,)),
            pltpu.SemaphoreType.DMA((N_DEV,)),
            pltpu.SemaphoreType.DMA((N_DEV,)),
            pltpu.SemaphoreType.DMA((N_DEV,)),
            pltpu.SemaphoreType.REGULAR((N_DEV,)),
            pltpu.SemaphoreType.DMA((2,)),
        ],
        compiler_params=pltpu.CompilerParams(
            collective_id=None if NOCOMM else 0,
            vmem_limit_bytes=100 * 1024 * 1024,
        ),
    )(x, w_mat)
